# Initial kernel scaffold; baseline (speedup 1.0000x reference)
import functools

import jax
import jax.numpy as jnp
from jax import lax
from jax.experimental import pallas as pl
from jax.experimental.pallas import tpu as pltpu
from jax.experimental.pallas import tpu_sc as plsc

_N = 10000
_E = 160000
_D = 256
_H = 128
_NPAD = 10240
_CHUNK = 128

_NC = 2
_NS = 16
_NW = _NC * _NS

_E_PAD = 163840
_DEG_CH = _E_PAD // _NW // _CHUNK
_SEG_CH = _E_PAD // _NS // _CHUNK
_ROWS_PER_TILE = _NPAD // _NS

_mesh = plsc.VectorSubcoreMesh(core_axis_name="c", subcore_axis_name="s")


@functools.partial(
    pl.kernel,
    mesh=_mesh,
    out_type=jax.ShapeDtypeStruct((_NC, _NPAD), jnp.float32),
    scratch_types=[
        pltpu.VMEM((_CHUNK,), jnp.float32),
        pltpu.VMEM((_DEG_CH, _CHUNK), jnp.int32),
        pltpu.VMEM_SHARED((_NPAD,), jnp.float32),
        pltpu.SemaphoreType.DMA,
    ],
)
def _deg_kernel(dst_hbm, zeros_hbm, ones_hbm, out_hbm, ones_v, idx_v, hist_sh, sem):
    c = lax.axis_index("c")
    s = lax.axis_index("s")
    w = s * _NC + c
    pltpu.sync_copy(ones_hbm, ones_v)
    pltpu.sync_copy(dst_hbm.at[pl.ds(w * _DEG_CH, _DEG_CH)], idx_v)

    @pl.when(s == 0)
    def _():
        pltpu.sync_copy(zeros_hbm, hist_sh)

    plsc.subcore_barrier()

    def body(j, carry):
        pltpu.sync_copy(ones_v, hist_sh.at[idx_v.at[j]], add=True)
        return carry

    lax.fori_loop(0, _DEG_CH, body, 0)
    plsc.subcore_barrier()

    @pl.when(s == 0)
    def _():
        pltpu.sync_copy(hist_sh, out_hbm.at[c])


@functools.partial(
    pl.kernel,
    mesh=_mesh,
    out_type=jax.ShapeDtypeStruct((_NC, _NPAD, _H), jnp.float32),
    scratch_types=[
        pltpu.VMEM((_SEG_CH, _CHUNK), jnp.int32),
        pltpu.VMEM((_SEG_CH, _CHUNK), jnp.int32),
        pltpu.VMEM((_CHUNK, _H), jnp.float32),
        pltpu.VMEM_SHARED((_NPAD, _H), jnp.float32),
        pltpu.SemaphoreType.DMA,
    ],
)
def _seg_kernel(y0, y1, src_hbm, dst_hbm, zrow_hbm, out_hbm,
                src_v, dst_v, rows_v, seg_sh, sem):
    c = lax.axis_index("c")
    s = lax.axis_index("s")
    pltpu.sync_copy(src_hbm.at[pl.ds(s * _SEG_CH, _SEG_CH)], src_v)
    pltpu.sync_copy(dst_hbm.at[pl.ds(s * _SEG_CH, _SEG_CH)], dst_v)
    pltpu.sync_copy(zrow_hbm, seg_sh.at[pl.ds(s * _ROWS_PER_TILE, _ROWS_PER_TILE)])
    plsc.subcore_barrier()

    def run(y_hbm):
        def body(j, carry):
            pltpu.async_copy(y_hbm.at[src_v.at[j]], rows_v, sem).wait()
            pltpu.sync_copy(rows_v, seg_sh.at[dst_v.at[j]], add=True)
            return carry
        lax.fori_loop(0, _SEG_CH, body, 0)

    @pl.when(c == 0)
    def _():
        run(y0)

    @pl.when(c == 1)
    def _():
        run(y1)

    plsc.subcore_barrier()
    pltpu.sync_copy(
        seg_sh.at[pl.ds(s * _ROWS_PER_TILE, _ROWS_PER_TILE)],
        out_hbm.at[c, pl.ds(s * _ROWS_PER_TILE, _ROWS_PER_TILE)],
    )


_BM = 1000


def _y_body(x_ref, w_ref, dp_ref, y0_ref, y1_ref):
    xw = jnp.dot(x_ref[...], w_ref[...], preferred_element_type=jnp.float32)
    deg = dp_ref[0] + dp_ref[1] + 1.0
    dinv = lax.rsqrt(deg)[:, None]
    y = xw * dinv
    y0_ref[...] = y[:, :_H]
    y1_ref[...] = y[:, _H:]


def _y_call(x, w1, deg_parts):
    return pl.pallas_call(
        _y_body,
        grid=(_N // _BM,),
        in_specs=[
            pl.BlockSpec((_BM, _D), lambda i: (i, 0)),
            pl.BlockSpec((_D, _D), lambda i: (0, 0)),
            pl.BlockSpec((_NC, _BM), lambda i: (0, i)),
        ],
        out_specs=[
            pl.BlockSpec((_BM, _H), lambda i: (i, 0)),
            pl.BlockSpec((_BM, _H), lambda i: (i, 0)),
        ],
        out_shape=[
            jax.ShapeDtypeStruct((_N, _H), jnp.float32),
            jax.ShapeDtypeStruct((_N, _H), jnp.float32),
        ],
    )(x, w1, deg_parts)


def _epi_body(seg_ref, y0_ref, y1_ref, dp_ref, b1_ref, wl_ref, bl_ref,
              h_ref, out_ref, acc_ref):
    i = pl.program_id(0)
    deg = dp_ref[0] + dp_ref[1] + 1.0
    dinv = lax.rsqrt(deg)[:, None]
    h0 = jnp.maximum(dinv * (seg_ref[0] + y0_ref[...]) + b1_ref[0, :_H], 0.0)
    h1 = jnp.maximum(dinv * (seg_ref[1] + y1_ref[...]) + b1_ref[0, _H:], 0.0)
    h_ref[:, :_H] = h0
    h_ref[:, _H:] = h1

    @pl.when(i == 0)
    def _():
        acc_ref[...] = jnp.zeros_like(acc_ref)

    acc_ref[0, :_H] += jnp.sum(h0, axis=0)
    acc_ref[0, _H:] += jnp.sum(h1, axis=0)

    @pl.when(i == pl.num_programs(0) - 1)
    def _():
        mean = acc_ref[...] * (1.0 / _N)
        out_ref[...] = lax.dot_general(
            mean, wl_ref[...], (((1,), (1,)), ((), ()))) + bl_ref[...]


def _epi_call(seg, y0, y1, deg_parts, b1, wl, bl):
    return pl.pallas_call(
        _epi_body,
        grid=(_N // _BM,),
        in_specs=[
            pl.BlockSpec((_NC, _BM, _H), lambda i: (0, i, 0)),
            pl.BlockSpec((_BM, _H), lambda i: (i, 0)),
            pl.BlockSpec((_BM, _H), lambda i: (i, 0)),
            pl.BlockSpec((_NC, _BM), lambda i: (0, i)),
            pl.BlockSpec((1, _D), lambda i: (0, 0)),
            pl.BlockSpec((_D, _D), lambda i: (0, 0)),
            pl.BlockSpec((1, _D), lambda i: (0, 0)),
        ],
        out_specs=[
            pl.BlockSpec((_BM, _D), lambda i: (i, 0)),
            pl.BlockSpec((1, _D), lambda i: (0, 0)),
        ],
        out_shape=[
            jax.ShapeDtypeStruct((_N, _D), jnp.float32),
            jax.ShapeDtypeStruct((1, _D), jnp.float32),
        ],
        scratch_shapes=[pltpu.VMEM((1, _D), jnp.float32)],
    )(seg, y0, y1, deg_parts, b1, wl, bl)


def kernel(graph_x, graph_edge, W1, b1, Wl, bl):
    src = graph_edge[0]
    dst = graph_edge[1]
    pad = _E_PAD - _E
    src_p = jnp.concatenate([src, jnp.zeros((pad,), jnp.int32)])
    dst_p = jnp.concatenate([dst, jnp.full((pad,), _N, jnp.int32)])
    src2d = src_p.reshape(_E_PAD // _CHUNK, _CHUNK)
    dst2d = dst_p.reshape(_E_PAD // _CHUNK, _CHUNK)

    zeros_n = jnp.zeros((_NPAD,), jnp.float32)
    ones_c = jnp.ones((_CHUNK,), jnp.float32)
    zrow = jnp.zeros((_ROWS_PER_TILE, _H), jnp.float32)

    deg_parts = _deg_kernel(dst2d, zeros_n, ones_c)
    y0, y1 = _y_call(graph_x, W1, deg_parts)
    seg = _seg_kernel(y0, y1, src2d, dst2d, zrow)
    h, out = _epi_call(seg, y0, y1, deg_parts,
                       b1.reshape(1, _D), Wl, bl.reshape(1, _D))
    return (h, out)


# speedup vs baseline: 8.9947x; 8.9947x over previous
import functools

import jax
import jax.numpy as jnp
from jax import lax
from jax.experimental import pallas as pl
from jax.experimental.pallas import tpu as pltpu
from jax.experimental.pallas import tpu_sc as plsc

_N = 10000
_E = 160000
_D = 256
_H = 128
_NPAD = 10240
_CHUNK = 128

_NC = 2
_NS = 16
_NW = _NC * _NS

_E_PAD = 163840
_DEG_CH = _E_PAD // _NW // _CHUNK
_SEG_CH = _E_PAD // _NS // _CHUNK
_ROWS_PER_TILE = _NPAD // _NS

_mesh = plsc.VectorSubcoreMesh(core_axis_name="c", subcore_axis_name="s")


@functools.partial(
    pl.kernel,
    mesh=_mesh,
    out_type=jax.ShapeDtypeStruct((_NC, _NPAD), jnp.float32),
    scratch_types=[
        pltpu.VMEM((_CHUNK,), jnp.float32),
        pltpu.VMEM((_DEG_CH, _CHUNK), jnp.int32),
        pltpu.VMEM_SHARED((_NPAD,), jnp.float32),
        pltpu.SemaphoreType.DMA,
    ],
)
def _deg_kernel(dst_hbm, zeros_hbm, ones_hbm, out_hbm, ones_v, idx_v, hist_sh, sem):
    c = lax.axis_index("c")
    s = lax.axis_index("s")
    w = s * _NC + c
    pltpu.sync_copy(ones_hbm, ones_v)
    pltpu.sync_copy(dst_hbm.at[pl.ds(w * _DEG_CH, _DEG_CH)], idx_v)

    @pl.when(s == 0)
    def _():
        pltpu.sync_copy(zeros_hbm, hist_sh)

    plsc.subcore_barrier()

    def body(j, carry):
        pltpu.sync_copy(ones_v, hist_sh.at[idx_v.at[j]], add=True)
        return carry

    lax.fori_loop(0, _DEG_CH, body, 0)
    plsc.subcore_barrier()

    @pl.when(s == 0)
    def _():
        pltpu.sync_copy(hist_sh, out_hbm.at[c])


@functools.partial(
    pl.kernel,
    mesh=_mesh,
    out_type=jax.ShapeDtypeStruct((_NC, _NPAD, _H), jnp.float32),
    scratch_types=[
        pltpu.VMEM((_SEG_CH, _CHUNK), jnp.int32),
        pltpu.VMEM((_SEG_CH, _CHUNK), jnp.int32),
        pltpu.VMEM((_CHUNK, _H), jnp.float32),
        pltpu.VMEM_SHARED((_NPAD, _H), jnp.float32),
        pltpu.SemaphoreType.DMA,
    ],
)
def _seg_kernel(y0, y1, src_hbm, dst_hbm, zrow_hbm, out_hbm,
                src_v, dst_v, rows_v, seg_sh, sem):
    c = lax.axis_index("c")
    s = lax.axis_index("s")
    pltpu.sync_copy(src_hbm.at[pl.ds(s * _SEG_CH, _SEG_CH)], src_v)
    pltpu.sync_copy(dst_hbm.at[pl.ds(s * _SEG_CH, _SEG_CH)], dst_v)
    pltpu.sync_copy(zrow_hbm, seg_sh.at[pl.ds(s * _ROWS_PER_TILE, _ROWS_PER_TILE)])
    plsc.subcore_barrier()

    def run(y_hbm):
        def body(j, carry):
            pltpu.async_copy(y_hbm.at[src_v.at[j]], rows_v, sem).wait()
            pltpu.sync_copy(rows_v, seg_sh.at[dst_v.at[j]], add=True)
            return carry
        lax.fori_loop(0, _SEG_CH, body, 0)

    @pl.when(c == 0)
    def _():
        run(y0)

    @pl.when(c == 1)
    def _():
        run(y1)

    plsc.subcore_barrier()
    pltpu.sync_copy(
        seg_sh.at[pl.ds(s * _ROWS_PER_TILE, _ROWS_PER_TILE)],
        out_hbm.at[c, pl.ds(s * _ROWS_PER_TILE, _ROWS_PER_TILE)],
    )


_BM = 1000


def _y_body(x_ref, w_ref, dp_ref, y0_ref, y1_ref):
    xw = jnp.dot(x_ref[...], w_ref[...], preferred_element_type=jnp.float32)
    deg = dp_ref[:, 0] + dp_ref[:, 1] + 1.0
    dinv = lax.rsqrt(deg)[:, None]
    y = xw * dinv
    y0_ref[...] = y[:, :_H]
    y1_ref[...] = y[:, _H:]


def _y_call(x, w1, deg_parts):
    return pl.pallas_call(
        _y_body,
        grid=(_N // _BM,),
        in_specs=[
            pl.BlockSpec((_BM, _D), lambda i: (i, 0)),
            pl.BlockSpec((_D, _D), lambda i: (0, 0)),
            pl.BlockSpec((_BM, _NC), lambda i: (i, 0)),
        ],
        out_specs=[
            pl.BlockSpec((_BM, _H), lambda i: (i, 0)),
            pl.BlockSpec((_BM, _H), lambda i: (i, 0)),
        ],
        out_shape=[
            jax.ShapeDtypeStruct((_N, _H), jnp.float32),
            jax.ShapeDtypeStruct((_N, _H), jnp.float32),
        ],
    )(x, w1, deg_parts)


def _epi_body(seg_ref, y0_ref, y1_ref, dp_ref, b1_ref, wl_ref, bl_ref,
              h_ref, out_ref, acc_ref):
    i = pl.program_id(0)
    deg = dp_ref[:, 0] + dp_ref[:, 1] + 1.0
    dinv = lax.rsqrt(deg)[:, None]
    h0 = jnp.maximum(dinv * (seg_ref[0] + y0_ref[...]) + b1_ref[0, :_H], 0.0)
    h1 = jnp.maximum(dinv * (seg_ref[1] + y1_ref[...]) + b1_ref[0, _H:], 0.0)
    h_ref[:, :_H] = h0
    h_ref[:, _H:] = h1

    @pl.when(i == 0)
    def _():
        acc_ref[...] = jnp.zeros_like(acc_ref)

    acc_ref[0, :_H] += jnp.sum(h0, axis=0)
    acc_ref[0, _H:] += jnp.sum(h1, axis=0)

    @pl.when(i == pl.num_programs(0) - 1)
    def _():
        mean = acc_ref[...] * (1.0 / _N)
        out_ref[...] = lax.dot_general(
            mean, wl_ref[...], (((1,), (1,)), ((), ()))) + bl_ref[...]


def _epi_call(seg, y0, y1, deg_parts, b1, wl, bl):
    return pl.pallas_call(
        _epi_body,
        grid=(_N // _BM,),
        in_specs=[
            pl.BlockSpec((_NC, _BM, _H), lambda i: (0, i, 0)),
            pl.BlockSpec((_BM, _H), lambda i: (i, 0)),
            pl.BlockSpec((_BM, _H), lambda i: (i, 0)),
            pl.BlockSpec((_BM, _NC), lambda i: (i, 0)),
            pl.BlockSpec((1, _D), lambda i: (0, 0)),
            pl.BlockSpec((_D, _D), lambda i: (0, 0)),
            pl.BlockSpec((1, _D), lambda i: (0, 0)),
        ],
        out_specs=[
            pl.BlockSpec((_BM, _D), lambda i: (i, 0)),
            pl.BlockSpec((1, _D), lambda i: (0, 0)),
        ],
        out_shape=[
            jax.ShapeDtypeStruct((_N, _D), jnp.float32),
            jax.ShapeDtypeStruct((1, _D), jnp.float32),
        ],
        scratch_shapes=[pltpu.VMEM((1, _D), jnp.float32)],
    )(seg, y0, y1, deg_parts, b1, wl, bl)


def kernel(graph_x, graph_edge, W1, b1, Wl, bl):
    src = graph_edge[0]
    dst = graph_edge[1]
    pad = _E_PAD - _E
    src_p = jnp.concatenate([src, jnp.zeros((pad,), jnp.int32)])
    dst_p = jnp.concatenate([dst, jnp.full((pad,), _N, jnp.int32)])
    src2d = src_p.reshape(_E_PAD // _CHUNK, _CHUNK)
    dst2d = dst_p.reshape(_E_PAD // _CHUNK, _CHUNK)

    zeros_n = jnp.zeros((_NPAD,), jnp.float32)
    ones_c = jnp.ones((_CHUNK,), jnp.float32)
    zrow = jnp.zeros((_ROWS_PER_TILE, _H), jnp.float32)

    deg_parts = _deg_kernel(dst2d, zeros_n, ones_c).T
    y0, y1 = _y_call(graph_x, W1, deg_parts)
    seg = _seg_kernel(y0, y1, src2d, dst2d, zrow)
    h, out = _epi_call(seg, y0, y1, deg_parts,
                       b1.reshape(1, _D), Wl, bl.reshape(1, _D))
    return (h, out)


# speedup vs baseline: 9.9088x; 1.1016x over previous
import functools

import jax
import jax.numpy as jnp
from jax import lax
from jax.experimental import pallas as pl
from jax.experimental.pallas import tpu as pltpu
from jax.experimental.pallas import tpu_sc as plsc

_N = 10000
_E = 160000
_D = 256
_H = 128
_NPAD = 10240
_CHUNK = 128

_NC = 2
_NS = 16
_NW = _NC * _NS

_E_PAD = 163840
_DEG_CH = _E_PAD // _NW // _CHUNK
_SEG_CH = _E_PAD // _NS // _CHUNK
_SEG_PASSES = 2
_PASS_CH = _SEG_CH // _SEG_PASSES
_ROWS_PER_TILE = _NPAD // _NS

_mesh = plsc.VectorSubcoreMesh(core_axis_name="c", subcore_axis_name="s")


@functools.partial(
    pl.kernel,
    mesh=_mesh,
    out_type=jax.ShapeDtypeStruct((_NC, _NPAD), jnp.float32),
    scratch_types=[
        pltpu.VMEM((_CHUNK,), jnp.float32),
        pltpu.VMEM((_DEG_CH, _CHUNK), jnp.int32),
        pltpu.VMEM_SHARED((_NPAD,), jnp.float32),
        pltpu.SemaphoreType.DMA,
    ],
)
def _deg_kernel(dst_hbm, zeros_hbm, ones_hbm, out_hbm, ones_v, idx_v, hist_sh, sem):
    c = lax.axis_index("c")
    s = lax.axis_index("s")
    w = s * _NC + c
    pltpu.sync_copy(ones_hbm, ones_v)
    pltpu.sync_copy(dst_hbm.at[pl.ds(w * _DEG_CH, _DEG_CH)], idx_v)

    @pl.when(s == 0)
    def _():
        pltpu.sync_copy(zeros_hbm, hist_sh)

    plsc.subcore_barrier()

    def fire(j, carry):
        pltpu.async_copy(ones_v, hist_sh.at[idx_v.at[j]], sem, add=True)
        return carry

    lax.fori_loop(0, _DEG_CH, fire, 0)

    def drain(j, carry):
        pltpu.make_async_copy(ones_v, hist_sh.at[idx_v.at[0]], sem).wait()
        return carry

    lax.fori_loop(0, _DEG_CH, drain, 0)
    plsc.subcore_barrier()

    @pl.when(s == 0)
    def _():
        pltpu.sync_copy(hist_sh, out_hbm.at[c])


@functools.partial(
    pl.kernel,
    mesh=_mesh,
    out_type=jax.ShapeDtypeStruct((_NC, _NPAD, _H), jnp.float32),
    scratch_types=[
        pltpu.VMEM((_PASS_CH, _CHUNK), jnp.int32),
        pltpu.VMEM((_PASS_CH, _CHUNK), jnp.int32),
        pltpu.VMEM((_CHUNK, _H), jnp.float32),
        pltpu.VMEM((_CHUNK, _H), jnp.float32),
        pltpu.VMEM_SHARED((_NPAD, _H), jnp.float32),
        pltpu.SemaphoreType.DMA,
        pltpu.SemaphoreType.DMA,
        pltpu.SemaphoreType.DMA,
        pltpu.SemaphoreType.DMA,
    ],
)
def _seg_kernel(y0, y1, src_hbm, dst_hbm, zrow_hbm, out_hbm,
                src_v, dst_v, rows_a, rows_b, seg_sh,
                sem_a, sem_b, sem_sa, sem_sb):
    c = lax.axis_index("c")
    s = lax.axis_index("s")
    pltpu.sync_copy(zrow_hbm, seg_sh.at[pl.ds(s * _ROWS_PER_TILE, _ROWS_PER_TILE)])
    plsc.subcore_barrier()

    def run(y_hbm):
        def g_fire(j, buf, sem):
            pltpu.async_copy(y_hbm.at[src_v.at[j]], buf, sem)

        def g_wait(buf, sem):
            pltpu.make_async_copy(y_hbm.at[src_v.at[0]], buf, sem).wait()

        def s_fire(j, buf, sem):
            pltpu.async_copy(buf, seg_sh.at[dst_v.at[j]], sem, add=True)

        def s_drain(buf, sem):
            pltpu.make_async_copy(buf, seg_sh.at[dst_v.at[0]], sem).wait()

        for p in range(_SEG_PASSES):
            base = s * _SEG_CH + p * _PASS_CH
            pltpu.sync_copy(src_hbm.at[pl.ds(base, _PASS_CH)], src_v)
            pltpu.sync_copy(dst_hbm.at[pl.ds(base, _PASS_CH)], dst_v)
            g_fire(0, rows_a, sem_a)

            def body(i, carry):
                j0 = 2 * i
                j1 = j0 + 1
                g_wait(rows_a, sem_a)
                s_fire(j0, rows_a, sem_sa)

                @pl.when(i > 0)
                def _():
                    s_drain(rows_b, sem_sb)

                g_fire(j1, rows_b, sem_b)
                g_wait(rows_b, sem_b)
                s_fire(j1, rows_b, sem_sb)
                s_drain(rows_a, sem_sa)

                @pl.when(j1 + 1 < _PASS_CH)
                def _():
                    g_fire(j1 + 1, rows_a, sem_a)
                return carry

            lax.fori_loop(0, _PASS_CH // 2, body, 0)
            s_drain(rows_b, sem_sb)

    @pl.when(c == 0)
    def _():
        run(y0)

    @pl.when(c == 1)
    def _():
        run(y1)

    plsc.subcore_barrier()
    pltpu.sync_copy(
        seg_sh.at[pl.ds(s * _ROWS_PER_TILE, _ROWS_PER_TILE)],
        out_hbm.at[c, pl.ds(s * _ROWS_PER_TILE, _ROWS_PER_TILE)],
    )


_BM = 1000


def _y_body(x_ref, w_ref, dp_ref, y0_ref, y1_ref):
    xw = jnp.dot(x_ref[...], w_ref[...], preferred_element_type=jnp.float32)
    deg = dp_ref[:, 0] + dp_ref[:, 1] + 1.0
    dinv = lax.rsqrt(deg)[:, None]
    y = xw * dinv
    y0_ref[...] = y[:, :_H]
    y1_ref[...] = y[:, _H:]


def _y_call(x, w1, deg_parts):
    return pl.pallas_call(
        _y_body,
        grid=(_N // _BM,),
        in_specs=[
            pl.BlockSpec((_BM, _D), lambda i: (i, 0)),
            pl.BlockSpec((_D, _D), lambda i: (0, 0)),
            pl.BlockSpec((_BM, _NC), lambda i: (i, 0)),
        ],
        out_specs=[
            pl.BlockSpec((_BM, _H), lambda i: (i, 0)),
            pl.BlockSpec((_BM, _H), lambda i: (i, 0)),
        ],
        out_shape=[
            jax.ShapeDtypeStruct((_N, _H), jnp.float32),
            jax.ShapeDtypeStruct((_N, _H), jnp.float32),
        ],
    )(x, w1, deg_parts)


def _epi_body(seg_ref, y0_ref, y1_ref, dp_ref, b1_ref, wl_ref, bl_ref,
              h_ref, out_ref, acc_ref):
    i = pl.program_id(0)
    deg = dp_ref[:, 0] + dp_ref[:, 1] + 1.0
    dinv = lax.rsqrt(deg)[:, None]
    h0 = jnp.maximum(dinv * (seg_ref[0] + y0_ref[...]) + b1_ref[0, :_H], 0.0)
    h1 = jnp.maximum(dinv * (seg_ref[1] + y1_ref[...]) + b1_ref[0, _H:], 0.0)
    h_ref[:, :_H] = h0
    h_ref[:, _H:] = h1

    @pl.when(i == 0)
    def _():
        acc_ref[...] = jnp.zeros_like(acc_ref)

    acc_ref[0, :_H] += jnp.sum(h0, axis=0)
    acc_ref[0, _H:] += jnp.sum(h1, axis=0)

    @pl.when(i == pl.num_programs(0) - 1)
    def _():
        mean = acc_ref[...] * (1.0 / _N)
        out_ref[...] = lax.dot_general(
            mean, wl_ref[...], (((1,), (1,)), ((), ()))) + bl_ref[...]


def _epi_call(seg, y0, y1, deg_parts, b1, wl, bl):
    return pl.pallas_call(
        _epi_body,
        grid=(_N // _BM,),
        in_specs=[
            pl.BlockSpec((_NC, _BM, _H), lambda i: (0, i, 0)),
            pl.BlockSpec((_BM, _H), lambda i: (i, 0)),
            pl.BlockSpec((_BM, _H), lambda i: (i, 0)),
            pl.BlockSpec((_BM, _NC), lambda i: (i, 0)),
            pl.BlockSpec((1, _D), lambda i: (0, 0)),
            pl.BlockSpec((_D, _D), lambda i: (0, 0)),
            pl.BlockSpec((1, _D), lambda i: (0, 0)),
        ],
        out_specs=[
            pl.BlockSpec((_BM, _D), lambda i: (i, 0)),
            pl.BlockSpec((1, _D), lambda i: (0, 0)),
        ],
        out_shape=[
            jax.ShapeDtypeStruct((_N, _D), jnp.float32),
            jax.ShapeDtypeStruct((1, _D), jnp.float32),
        ],
        scratch_shapes=[pltpu.VMEM((1, _D), jnp.float32)],
    )(seg, y0, y1, deg_parts, b1, wl, bl)


def kernel(graph_x, graph_edge, W1, b1, Wl, bl):
    src = graph_edge[0]
    dst = graph_edge[1]
    pad = _E_PAD - _E
    src_p = jnp.concatenate([src, jnp.zeros((pad,), jnp.int32)])
    dst_p = jnp.concatenate([dst, jnp.full((pad,), _N, jnp.int32)])
    src2d = src_p.reshape(_E_PAD // _CHUNK, _CHUNK)
    dst2d = dst_p.reshape(_E_PAD // _CHUNK, _CHUNK)

    zeros_n = jnp.zeros((_NPAD,), jnp.float32)
    ones_c = jnp.ones((_CHUNK,), jnp.float32)
    zrow = jnp.zeros((_ROWS_PER_TILE, _H), jnp.float32)

    deg_parts = _deg_kernel(dst2d, zeros_n, ones_c).T
    y0, y1 = _y_call(graph_x, W1, deg_parts)
    seg = _seg_kernel(y0, y1, src2d, dst2d, zrow)
    h, out = _epi_call(seg, y0, y1, deg_parts,
                       b1.reshape(1, _D), Wl, bl.reshape(1, _D))
    return (h, out)


# speedup vs baseline: 10.0686x; 1.0161x over previous
import functools

import jax
import jax.numpy as jnp
from jax import lax
from jax.experimental import pallas as pl
from jax.experimental.pallas import tpu as pltpu
from jax.experimental.pallas import tpu_sc as plsc

_N = 10000
_E = 160000
_D = 256
_H = 128
_NPAD = 10240
_CHUNK = 128
_SCHUNK = 64

_NC = 2
_NS = 16
_NW = _NC * _NS

_E_PAD = 163840
_DEG_CH = _E_PAD // _NW // _CHUNK
_SEG_CH = _E_PAD // _NS // _SCHUNK
_SEG_PASSES = 4
_PASS_CH = _SEG_CH // _SEG_PASSES
_NBUF = 4
_ROWS_PER_TILE = _NPAD // _NS

_mesh = plsc.VectorSubcoreMesh(core_axis_name="c", subcore_axis_name="s")


@functools.partial(
    pl.kernel,
    mesh=_mesh,
    out_type=jax.ShapeDtypeStruct((_NC, _NPAD), jnp.float32),
    scratch_types=[
        pltpu.VMEM((_CHUNK,), jnp.float32),
        pltpu.VMEM((_DEG_CH, _CHUNK), jnp.int32),
        pltpu.VMEM_SHARED((_NPAD,), jnp.float32),
        pltpu.SemaphoreType.DMA,
    ],
)
def _deg_kernel(dst_hbm, zeros_hbm, ones_hbm, out_hbm, ones_v, idx_v, hist_sh, sem):
    c = lax.axis_index("c")
    s = lax.axis_index("s")
    w = s * _NC + c
    pltpu.sync_copy(ones_hbm, ones_v)
    pltpu.sync_copy(dst_hbm.at[pl.ds(w * _DEG_CH, _DEG_CH)], idx_v)

    @pl.when(s == 0)
    def _():
        pltpu.sync_copy(zeros_hbm, hist_sh)

    plsc.subcore_barrier()

    def fire(j, carry):
        pltpu.async_copy(ones_v, hist_sh.at[idx_v.at[j]], sem, add=True)
        return carry

    lax.fori_loop(0, _DEG_CH, fire, 0)

    def drain(j, carry):
        pltpu.make_async_copy(ones_v, hist_sh.at[idx_v.at[0]], sem).wait()
        return carry

    lax.fori_loop(0, _DEG_CH, drain, 0)
    plsc.subcore_barrier()

    @pl.when(s == 0)
    def _():
        pltpu.sync_copy(hist_sh, out_hbm.at[c])


@functools.partial(
    pl.kernel,
    mesh=_mesh,
    out_type=jax.ShapeDtypeStruct((_NC, _NPAD, _H), jnp.float32),
    scratch_types=[
        pltpu.VMEM((_PASS_CH, _SCHUNK), jnp.int32),
        pltpu.VMEM((_PASS_CH, _SCHUNK), jnp.int32),
        pltpu.VMEM((_SCHUNK, _H), jnp.float32),
        pltpu.VMEM((_SCHUNK, _H), jnp.float32),
        pltpu.VMEM((_SCHUNK, _H), jnp.float32),
        pltpu.VMEM((_SCHUNK, _H), jnp.float32),
        pltpu.VMEM_SHARED((_NPAD, _H), jnp.float32),
        pltpu.SemaphoreType.DMA,
        pltpu.SemaphoreType.DMA,
        pltpu.SemaphoreType.DMA,
        pltpu.SemaphoreType.DMA,
        pltpu.SemaphoreType.DMA,
        pltpu.SemaphoreType.DMA,
        pltpu.SemaphoreType.DMA,
        pltpu.SemaphoreType.DMA,
    ],
)
def _seg_kernel(y0, y1, src_hbm, dst_hbm, zrow_hbm, out_hbm,
                src_v, dst_v, b0, b1, b2, b3, seg_sh,
                g0, g1, g2, g3, s0, s1, s2, s3):
    c = lax.axis_index("c")
    s = lax.axis_index("s")
    bufs = (b0, b1, b2, b3)
    gsems = (g0, g1, g2, g3)
    ssems = (s0, s1, s2, s3)
    pltpu.sync_copy(zrow_hbm, seg_sh.at[pl.ds(s * _ROWS_PER_TILE, _ROWS_PER_TILE)])
    plsc.subcore_barrier()

    def run(y_hbm):
        def g_fire(j, buf, sem):
            pltpu.async_copy(y_hbm.at[src_v.at[j]], buf, sem)

        def g_wait(buf, sem):
            pltpu.make_async_copy(y_hbm.at[src_v.at[0]], buf, sem).wait()

        def s_fire(j, buf, sem):
            pltpu.async_copy(buf, seg_sh.at[dst_v.at[j]], sem, add=True)

        def s_drain(buf, sem):
            pltpu.make_async_copy(buf, seg_sh.at[dst_v.at[0]], sem).wait()

        for p in range(_SEG_PASSES):
            base = s * _SEG_CH + p * _PASS_CH
            pltpu.sync_copy(src_hbm.at[pl.ds(base, _PASS_CH)], src_v)
            pltpu.sync_copy(dst_hbm.at[pl.ds(base, _PASS_CH)], dst_v)
            g_fire(0, b0, g0)
            g_fire(1, b1, g1)

            def body(i, carry):
                for k in range(_NBUF):
                    jj = _NBUF * i + k
                    q = (k + 2) % _NBUF
                    g_wait(bufs[k], gsems[k])
                    s_fire(jj, bufs[k], ssems[k])
                    jn = jj + 2

                    @pl.when(jn < _PASS_CH)
                    def _(jn=jn, q=q):
                        @pl.when(jn >= _NBUF)
                        def _():
                            s_drain(bufs[q], ssems[q])
                        g_fire(jn, bufs[q], gsems[q])
                return carry

            lax.fori_loop(0, _PASS_CH // _NBUF, body, 0)
            for k in range(_NBUF):
                s_drain(bufs[(k + 2) % _NBUF], ssems[(k + 2) % _NBUF])

    @pl.when(c == 0)
    def _():
        run(y0)

    @pl.when(c == 1)
    def _():
        run(y1)

    plsc.subcore_barrier()
    pltpu.sync_copy(
        seg_sh.at[pl.ds(s * _ROWS_PER_TILE, _ROWS_PER_TILE)],
        out_hbm.at[c, pl.ds(s * _ROWS_PER_TILE, _ROWS_PER_TILE)],
    )


_BM = 1000


def _y_body(x_ref, w_ref, dp_ref, y0_ref, y1_ref):
    xw = jnp.dot(x_ref[...], w_ref[...], preferred_element_type=jnp.float32)
    deg = dp_ref[:, 0] + dp_ref[:, 1] + 1.0
    dinv = lax.rsqrt(deg)[:, None]
    y = xw * dinv
    y0_ref[...] = y[:, :_H]
    y1_ref[...] = y[:, _H:]


def _y_call(x, w1, deg_parts):
    return pl.pallas_call(
        _y_body,
        grid=(_N // _BM,),
        in_specs=[
            pl.BlockSpec((_BM, _D), lambda i: (i, 0)),
            pl.BlockSpec((_D, _D), lambda i: (0, 0)),
            pl.BlockSpec((_BM, _NC), lambda i: (i, 0)),
        ],
        out_specs=[
            pl.BlockSpec((_BM, _H), lambda i: (i, 0)),
            pl.BlockSpec((_BM, _H), lambda i: (i, 0)),
        ],
        out_shape=[
            jax.ShapeDtypeStruct((_N, _H), jnp.float32),
            jax.ShapeDtypeStruct((_N, _H), jnp.float32),
        ],
    )(x, w1, deg_parts)


def _epi_body(seg_ref, y0_ref, y1_ref, dp_ref, b1_ref, wl_ref, bl_ref,
              h_ref, out_ref, acc_ref):
    i = pl.program_id(0)
    deg = dp_ref[:, 0] + dp_ref[:, 1] + 1.0
    dinv = lax.rsqrt(deg)[:, None]
    h0 = jnp.maximum(dinv * (seg_ref[0] + y0_ref[...]) + b1_ref[0, :_H], 0.0)
    h1 = jnp.maximum(dinv * (seg_ref[1] + y1_ref[...]) + b1_ref[0, _H:], 0.0)
    h_ref[:, :_H] = h0
    h_ref[:, _H:] = h1

    @pl.when(i == 0)
    def _():
        acc_ref[...] = jnp.zeros_like(acc_ref)

    acc_ref[0, :_H] += jnp.sum(h0, axis=0)
    acc_ref[0, _H:] += jnp.sum(h1, axis=0)

    @pl.when(i == pl.num_programs(0) - 1)
    def _():
        mean = acc_ref[...] * (1.0 / _N)
        out_ref[...] = lax.dot_general(
            mean, wl_ref[...], (((1,), (1,)), ((), ()))) + bl_ref[...]


def _epi_call(seg, y0, y1, deg_parts, b1, wl, bl):
    return pl.pallas_call(
        _epi_body,
        grid=(_N // _BM,),
        in_specs=[
            pl.BlockSpec((_NC, _BM, _H), lambda i: (0, i, 0)),
            pl.BlockSpec((_BM, _H), lambda i: (i, 0)),
            pl.BlockSpec((_BM, _H), lambda i: (i, 0)),
            pl.BlockSpec((_BM, _NC), lambda i: (i, 0)),
            pl.BlockSpec((1, _D), lambda i: (0, 0)),
            pl.BlockSpec((_D, _D), lambda i: (0, 0)),
            pl.BlockSpec((1, _D), lambda i: (0, 0)),
        ],
        out_specs=[
            pl.BlockSpec((_BM, _D), lambda i: (i, 0)),
            pl.BlockSpec((1, _D), lambda i: (0, 0)),
        ],
        out_shape=[
            jax.ShapeDtypeStruct((_N, _D), jnp.float32),
            jax.ShapeDtypeStruct((1, _D), jnp.float32),
        ],
        scratch_shapes=[pltpu.VMEM((1, _D), jnp.float32)],
    )(seg, y0, y1, deg_parts, b1, wl, bl)


def kernel(graph_x, graph_edge, W1, b1, Wl, bl):
    src = graph_edge[0]
    dst = graph_edge[1]
    pad = _E_PAD - _E
    src_p = jnp.concatenate([src, jnp.zeros((pad,), jnp.int32)])
    dst_p = jnp.concatenate([dst, jnp.full((pad,), _N, jnp.int32)])
    dst2d = dst_p.reshape(_E_PAD // _CHUNK, _CHUNK)
    srcs = src_p.reshape(_E_PAD // _SCHUNK, _SCHUNK)
    dsts = dst_p.reshape(_E_PAD // _SCHUNK, _SCHUNK)

    zeros_n = jnp.zeros((_NPAD,), jnp.float32)
    ones_c = jnp.ones((_CHUNK,), jnp.float32)
    zrow = jnp.zeros((_ROWS_PER_TILE, _H), jnp.float32)

    deg_parts = _deg_kernel(dst2d, zeros_n, ones_c).T
    y0, y1 = _y_call(graph_x, W1, deg_parts)
    seg = _seg_kernel(y0, y1, srcs, dsts, zrow)
    h, out = _epi_call(seg, y0, y1, deg_parts,
                       b1.reshape(1, _D), Wl, bl.reshape(1, _D))
    return (h, out)


# speedup vs baseline: 10.3901x; 1.0319x over previous
import functools

import jax
import jax.numpy as jnp
from jax import lax
from jax.experimental import pallas as pl
from jax.experimental.pallas import tpu as pltpu
from jax.experimental.pallas import tpu_sc as plsc

_N = 10000
_E = 160000
_D = 256
_H = 128
_NPAD = 10240
_CHUNK = 128
_SCHUNK = 64

_NC = 2
_NS = 16
_NW = _NC * _NS

_E_PAD = 163840
_DEG_CH = _E_PAD // _NW // _CHUNK
_SEG_CH = _E_PAD // _NS // _SCHUNK
_SEG_PASSES = 4
_PASS_CH = _SEG_CH // _SEG_PASSES
_NBUF = 4
_ROWS_PER_TILE = _NPAD // _NS

_mesh = plsc.VectorSubcoreMesh(core_axis_name="c", subcore_axis_name="s")


@functools.partial(
    pl.kernel,
    mesh=_mesh,
    out_type=jax.ShapeDtypeStruct((_NC, _NPAD), jnp.float32),
    scratch_types=[
        pltpu.VMEM((_CHUNK,), jnp.float32),
        pltpu.VMEM((_DEG_CH, _CHUNK), jnp.int32),
        pltpu.VMEM_SHARED((_NPAD,), jnp.float32),
        pltpu.SemaphoreType.DMA,
    ],
)
def _deg_kernel(dst_hbm, zeros_hbm, ones_hbm, out_hbm, ones_v, idx_v, hist_sh, sem):
    c = lax.axis_index("c")
    s = lax.axis_index("s")
    w = s * _NC + c
    pltpu.sync_copy(ones_hbm, ones_v)
    pltpu.sync_copy(dst_hbm.at[pl.ds(w * _DEG_CH, _DEG_CH)], idx_v)

    @pl.when(s == 0)
    def _():
        pltpu.sync_copy(zeros_hbm, hist_sh)

    plsc.subcore_barrier()

    def fire(j, carry):
        pltpu.async_copy(ones_v, hist_sh.at[idx_v.at[j]], sem, add=True)
        return carry

    lax.fori_loop(0, _DEG_CH, fire, 0)

    def drain(j, carry):
        pltpu.make_async_copy(ones_v, hist_sh.at[idx_v.at[0]], sem).wait()
        return carry

    lax.fori_loop(0, _DEG_CH, drain, 0)
    plsc.subcore_barrier()

    @pl.when(s == 0)
    def _():
        pltpu.sync_copy(hist_sh, out_hbm.at[c])


@functools.partial(
    pl.kernel,
    mesh=_mesh,
    out_type=jax.ShapeDtypeStruct((_NC, _NPAD, _H), jnp.float32),
    scratch_types=[
        pltpu.VMEM((_PASS_CH, _SCHUNK), jnp.int32),
        pltpu.VMEM((_PASS_CH, _SCHUNK), jnp.int32),
        pltpu.VMEM((_SCHUNK, _H), jnp.float32),
        pltpu.VMEM((_SCHUNK, _H), jnp.float32),
        pltpu.VMEM((_SCHUNK, _H), jnp.float32),
        pltpu.VMEM((_SCHUNK, _H), jnp.float32),
        pltpu.VMEM_SHARED((_NPAD, _H), jnp.float32),
        pltpu.SemaphoreType.DMA,
        pltpu.SemaphoreType.DMA,
        pltpu.SemaphoreType.DMA,
        pltpu.SemaphoreType.DMA,
        pltpu.SemaphoreType.DMA,
        pltpu.SemaphoreType.DMA,
        pltpu.SemaphoreType.DMA,
        pltpu.SemaphoreType.DMA,
    ],
)
def _seg_kernel(y0, y1, src_hbm, dst_hbm, zrow_hbm, out_hbm,
                src_v, dst_v, b0, b1, b2, b3, seg_sh,
                g0, g1, g2, g3, s0, s1, s2, s3):
    c = lax.axis_index("c")
    s = lax.axis_index("s")
    bufs = (b0, b1, b2, b3)
    gsems = (g0, g1, g2, g3)
    ssems = (s0, s1, s2, s3)
    pltpu.sync_copy(zrow_hbm, seg_sh.at[pl.ds(s * _ROWS_PER_TILE, _ROWS_PER_TILE)])
    plsc.subcore_barrier()

    def run(y_hbm):
        def g_fire(j, buf, sem):
            pltpu.async_copy(y_hbm.at[src_v.at[j]], buf, sem)

        def g_wait(buf, sem):
            pltpu.make_async_copy(y_hbm.at[src_v.at[0]], buf, sem).wait()

        def s_fire(j, buf, sem):
            pltpu.async_copy(buf, seg_sh.at[dst_v.at[j]], sem, add=True)

        def s_drain(buf, sem):
            pltpu.make_async_copy(buf, seg_sh.at[dst_v.at[0]], sem).wait()

        for p in range(_SEG_PASSES):
            base = s * _SEG_CH + p * _PASS_CH
            pltpu.sync_copy(src_hbm.at[pl.ds(base, _PASS_CH)], src_v)
            pltpu.sync_copy(dst_hbm.at[pl.ds(base, _PASS_CH)], dst_v)
            g_fire(0, b0, g0)
            g_fire(1, b1, g1)
            g_fire(2, b2, g2)

            def body(i, carry):
                for k in range(_NBUF):
                    jj = _NBUF * i + k
                    q = (k + 3) % _NBUF
                    jn = jj + 3

                    @pl.when(jn < _PASS_CH)
                    def _(jn=jn, q=q):
                        g_fire(jn, bufs[q], gsems[q])
                    g_wait(bufs[k], gsems[k])
                    pltpu.sync_copy(bufs[k], seg_sh.at[dst_v.at[jj]], add=True)
                return carry

            lax.fori_loop(0, _PASS_CH // _NBUF, body, 0)

    @pl.when(c == 0)
    def _():
        run(y0)

    @pl.when(c == 1)
    def _():
        run(y1)

    plsc.subcore_barrier()
    pltpu.sync_copy(
        seg_sh.at[pl.ds(s * _ROWS_PER_TILE, _ROWS_PER_TILE)],
        out_hbm.at[c, pl.ds(s * _ROWS_PER_TILE, _ROWS_PER_TILE)],
    )


_BM = 1000


def _y_body(x_ref, w_ref, dp_ref, y0_ref, y1_ref):
    xw = jnp.dot(x_ref[...], w_ref[...], preferred_element_type=jnp.float32)
    deg = dp_ref[:, 0] + dp_ref[:, 1] + 1.0
    dinv = lax.rsqrt(deg)[:, None]
    y = xw * dinv
    y0_ref[...] = y[:, :_H]
    y1_ref[...] = y[:, _H:]


def _y_call(x, w1, deg_parts):
    return pl.pallas_call(
        _y_body,
        grid=(_N // _BM,),
        in_specs=[
            pl.BlockSpec((_BM, _D), lambda i: (i, 0)),
            pl.BlockSpec((_D, _D), lambda i: (0, 0)),
            pl.BlockSpec((_BM, _NC), lambda i: (i, 0)),
        ],
        out_specs=[
            pl.BlockSpec((_BM, _H), lambda i: (i, 0)),
            pl.BlockSpec((_BM, _H), lambda i: (i, 0)),
        ],
        out_shape=[
            jax.ShapeDtypeStruct((_N, _H), jnp.float32),
            jax.ShapeDtypeStruct((_N, _H), jnp.float32),
        ],
    )(x, w1, deg_parts)


def _epi_body(seg_ref, y0_ref, y1_ref, dp_ref, b1_ref, wl_ref, bl_ref,
              h_ref, out_ref, acc_ref):
    i = pl.program_id(0)
    deg = dp_ref[:, 0] + dp_ref[:, 1] + 1.0
    dinv = lax.rsqrt(deg)[:, None]
    h0 = jnp.maximum(dinv * (seg_ref[0] + y0_ref[...]) + b1_ref[0, :_H], 0.0)
    h1 = jnp.maximum(dinv * (seg_ref[1] + y1_ref[...]) + b1_ref[0, _H:], 0.0)
    h_ref[:, :_H] = h0
    h_ref[:, _H:] = h1

    @pl.when(i == 0)
    def _():
        acc_ref[...] = jnp.zeros_like(acc_ref)

    acc_ref[0, :_H] += jnp.sum(h0, axis=0)
    acc_ref[0, _H:] += jnp.sum(h1, axis=0)

    @pl.when(i == pl.num_programs(0) - 1)
    def _():
        mean = acc_ref[...] * (1.0 / _N)
        out_ref[...] = lax.dot_general(
            mean, wl_ref[...], (((1,), (1,)), ((), ()))) + bl_ref[...]


def _epi_call(seg, y0, y1, deg_parts, b1, wl, bl):
    return pl.pallas_call(
        _epi_body,
        grid=(_N // _BM,),
        in_specs=[
            pl.BlockSpec((_NC, _BM, _H), lambda i: (0, i, 0)),
            pl.BlockSpec((_BM, _H), lambda i: (i, 0)),
            pl.BlockSpec((_BM, _H), lambda i: (i, 0)),
            pl.BlockSpec((_BM, _NC), lambda i: (i, 0)),
            pl.BlockSpec((1, _D), lambda i: (0, 0)),
            pl.BlockSpec((_D, _D), lambda i: (0, 0)),
            pl.BlockSpec((1, _D), lambda i: (0, 0)),
        ],
        out_specs=[
            pl.BlockSpec((_BM, _D), lambda i: (i, 0)),
            pl.BlockSpec((1, _D), lambda i: (0, 0)),
        ],
        out_shape=[
            jax.ShapeDtypeStruct((_N, _D), jnp.float32),
            jax.ShapeDtypeStruct((1, _D), jnp.float32),
        ],
        scratch_shapes=[pltpu.VMEM((1, _D), jnp.float32)],
    )(seg, y0, y1, deg_parts, b1, wl, bl)


def kernel(graph_x, graph_edge, W1, b1, Wl, bl):
    src = graph_edge[0]
    dst = graph_edge[1]
    pad = _E_PAD - _E
    src_p = jnp.concatenate([src, jnp.zeros((pad,), jnp.int32)])
    dst_p = jnp.concatenate([dst, jnp.full((pad,), _N, jnp.int32)])
    dst2d = dst_p.reshape(_E_PAD // _CHUNK, _CHUNK)
    srcs = src_p.reshape(_E_PAD // _SCHUNK, _SCHUNK)
    dsts = dst_p.reshape(_E_PAD // _SCHUNK, _SCHUNK)

    zeros_n = jnp.zeros((_NPAD,), jnp.float32)
    ones_c = jnp.ones((_CHUNK,), jnp.float32)
    zrow = jnp.zeros((_ROWS_PER_TILE, _H), jnp.float32)

    deg_parts = _deg_kernel(dst2d, zeros_n, ones_c).T
    y0, y1 = _y_call(graph_x, W1, deg_parts)
    seg = _seg_kernel(y0, y1, srcs, dsts, zrow)
    h, out = _epi_call(seg, y0, y1, deg_parts,
                       b1.reshape(1, _D), Wl, bl.reshape(1, _D))
    return (h, out)


# speedup vs baseline: 10.6766x; 1.0276x over previous
import functools

import jax
import jax.numpy as jnp
from jax import lax
from jax.experimental import pallas as pl
from jax.experimental.pallas import tpu as pltpu
from jax.experimental.pallas import tpu_sc as plsc

_N = 10000
_E = 160000
_D = 256
_H = 128
_NPAD = 10240
_CHUNK = 128
_SCHUNK = 64

_NC = 2
_NS = 16
_NW = _NC * _NS

_E_PAD = 163840
_DEG_CH = _E_PAD // _NW // _CHUNK
_SEG_CH = _E_PAD // _NS // _SCHUNK
_SEG_PASSES = 4
_PASS_CH = _SEG_CH // _SEG_PASSES
_NBUF = 4
_ROWS_PER_TILE = _NPAD // _NS

_mesh = plsc.VectorSubcoreMesh(core_axis_name="c", subcore_axis_name="s")


@functools.partial(
    pl.kernel,
    mesh=_mesh,
    out_type=jax.ShapeDtypeStruct((_NC, _NPAD), jnp.float32),
    scratch_types=[
        pltpu.VMEM((_CHUNK,), jnp.float32),
        pltpu.VMEM((_DEG_CH, _CHUNK), jnp.int32),
        pltpu.VMEM_SHARED((_NPAD,), jnp.float32),
        pltpu.SemaphoreType.DMA,
    ],
)
def _deg_kernel(dst_hbm, zeros_hbm, ones_hbm, out_hbm, ones_v, idx_v, hist_sh, sem):
    c = lax.axis_index("c")
    s = lax.axis_index("s")
    w = s * _NC + c
    pltpu.sync_copy(ones_hbm, ones_v)
    pltpu.sync_copy(dst_hbm.at[pl.ds(w * _DEG_CH, _DEG_CH)], idx_v)

    @pl.when(s == 0)
    def _():
        pltpu.sync_copy(zeros_hbm, hist_sh)

    plsc.subcore_barrier()

    def fire(j, carry):
        pltpu.async_copy(ones_v, hist_sh.at[idx_v.at[j]], sem, add=True)
        return carry

    lax.fori_loop(0, _DEG_CH, fire, 0)

    def drain(j, carry):
        pltpu.make_async_copy(ones_v, hist_sh.at[idx_v.at[0]], sem).wait()
        return carry

    lax.fori_loop(0, _DEG_CH, drain, 0)
    plsc.subcore_barrier()

    @pl.when(s == 0)
    def _():
        pltpu.sync_copy(hist_sh, out_hbm.at[c])


@functools.partial(
    pl.kernel,
    mesh=_mesh,
    out_type=jax.ShapeDtypeStruct((_NC, _NPAD, _H), jnp.float32),
    scratch_types=[
        pltpu.VMEM((_PASS_CH, _SCHUNK), jnp.int32),
        pltpu.VMEM((_PASS_CH, _SCHUNK), jnp.int32),
        pltpu.VMEM((_SCHUNK, _H), jnp.float32),
        pltpu.VMEM((_SCHUNK, _H), jnp.float32),
        pltpu.VMEM((_SCHUNK, _H), jnp.float32),
        pltpu.VMEM((_SCHUNK, _H), jnp.float32),
        pltpu.VMEM_SHARED((_NPAD, _H), jnp.float32),
        pltpu.SemaphoreType.DMA,
        pltpu.SemaphoreType.DMA,
        pltpu.SemaphoreType.DMA,
        pltpu.SemaphoreType.DMA,
        pltpu.SemaphoreType.DMA,
        pltpu.SemaphoreType.DMA,
        pltpu.SemaphoreType.DMA,
        pltpu.SemaphoreType.DMA,
    ],
)
def _seg_kernel(y0, y1, src_hbm, dst_hbm, zrow_hbm, out_hbm,
                src_v, dst_v, b0, b1, b2, b3, seg_sh,
                g0, g1, g2, g3, s0, s1, s2, s3):
    c = lax.axis_index("c")
    s = lax.axis_index("s")
    bufs = (b0, b1, b2, b3)
    gsems = (g0, g1, g2, g3)
    ssems = (s0, s1, s2, s3)
    pltpu.sync_copy(zrow_hbm, seg_sh.at[pl.ds(s * _ROWS_PER_TILE, _ROWS_PER_TILE)])
    plsc.subcore_barrier()

    def run(y_hbm):
        def g_fire(j, buf, sem):
            pltpu.async_copy(y_hbm.at[src_v.at[j]], buf, sem)

        def g_wait(buf, sem):
            pltpu.make_async_copy(y_hbm.at[src_v.at[0]], buf, sem).wait()

        def s_fire(j, buf, sem):
            pltpu.async_copy(buf, seg_sh.at[dst_v.at[j]], sem, add=True)

        def s_drain(buf, sem):
            pltpu.make_async_copy(buf, seg_sh.at[dst_v.at[0]], sem).wait()

        for p in range(_SEG_PASSES):
            base = s * _SEG_CH + p * _PASS_CH
            pltpu.sync_copy(src_hbm.at[pl.ds(base, _PASS_CH)], src_v)
            pltpu.sync_copy(dst_hbm.at[pl.ds(base, _PASS_CH)], dst_v)
            g_fire(0, b0, g0)
            g_fire(1, b1, g1)
            g_fire(2, b2, g2)

            def body(i, carry):
                for k in range(_NBUF):
                    jj = _NBUF * i + k
                    q = (k + 3) % _NBUF
                    jn = jj + 3

                    @pl.when(jn < _PASS_CH)
                    def _(jn=jn, q=q):
                        g_fire(jn, bufs[q], gsems[q])
                    g_wait(bufs[k], gsems[k])
                    pltpu.sync_copy(bufs[k], seg_sh.at[dst_v.at[jj]], add=True)
                return carry

            lax.fori_loop(0, _PASS_CH // _NBUF, body, 0)

    @pl.when(c == 0)
    def _():
        run(y0)

    @pl.when(c == 1)
    def _():
        run(y1)

    plsc.subcore_barrier()
    pltpu.sync_copy(
        seg_sh.at[pl.ds(s * _ROWS_PER_TILE, _ROWS_PER_TILE)],
        out_hbm.at[c, pl.ds(s * _ROWS_PER_TILE, _ROWS_PER_TILE)],
    )


_BM = 1000


def _mm_body(x_ref, w_ref, xw_ref):
    xw_ref[...] = jnp.dot(x_ref[...], w_ref[...],
                          preferred_element_type=jnp.float32)


def _mm_call(x, w1):
    return pl.pallas_call(
        _mm_body,
        grid=(_N // _BM,),
        in_specs=[
            pl.BlockSpec((_BM, _D), lambda i: (i, 0)),
            pl.BlockSpec((_D, _D), lambda i: (0, 0)),
        ],
        out_specs=pl.BlockSpec((_BM, _D), lambda i: (i, 0)),
        out_shape=jax.ShapeDtypeStruct((_N, _D), jnp.float32),
    )(x, w1)


def _y_body(xw_ref, dp_ref, y0_ref, y1_ref):
    deg = dp_ref[:, 0] + dp_ref[:, 1] + 1.0
    dinv = lax.rsqrt(deg)[:, None]
    y = xw_ref[...] * dinv
    y0_ref[...] = y[:, :_H]
    y1_ref[...] = y[:, _H:]


def _y_call(xw, deg_parts):
    return pl.pallas_call(
        _y_body,
        grid=(_N // _BM,),
        in_specs=[
            pl.BlockSpec((_BM, _D), lambda i: (i, 0)),
            pl.BlockSpec((_BM, _NC), lambda i: (i, 0)),
        ],
        out_specs=[
            pl.BlockSpec((_BM, _H), lambda i: (i, 0)),
            pl.BlockSpec((_BM, _H), lambda i: (i, 0)),
        ],
        out_shape=[
            jax.ShapeDtypeStruct((_N, _H), jnp.float32),
            jax.ShapeDtypeStruct((_N, _H), jnp.float32),
        ],
    )(xw, deg_parts)


def _epi_body(seg_ref, y0_ref, y1_ref, dp_ref, b1_ref, wl_ref, bl_ref,
              h_ref, out_ref, acc_ref):
    i = pl.program_id(0)
    deg = dp_ref[:, 0] + dp_ref[:, 1] + 1.0
    dinv = lax.rsqrt(deg)[:, None]
    h0 = jnp.maximum(dinv * (seg_ref[0] + y0_ref[...]) + b1_ref[0, :_H], 0.0)
    h1 = jnp.maximum(dinv * (seg_ref[1] + y1_ref[...]) + b1_ref[0, _H:], 0.0)
    h_ref[:, :_H] = h0
    h_ref[:, _H:] = h1

    @pl.when(i == 0)
    def _():
        acc_ref[...] = jnp.zeros_like(acc_ref)

    acc_ref[0, :_H] += jnp.sum(h0, axis=0)
    acc_ref[0, _H:] += jnp.sum(h1, axis=0)

    @pl.when(i == pl.num_programs(0) - 1)
    def _():
        mean = acc_ref[...] * (1.0 / _N)
        out_ref[...] = lax.dot_general(
            mean, wl_ref[...], (((1,), (1,)), ((), ()))) + bl_ref[...]


def _epi_call(seg, y0, y1, deg_parts, b1, wl, bl):
    return pl.pallas_call(
        _epi_body,
        grid=(_N // _BM,),
        in_specs=[
            pl.BlockSpec((_NC, _BM, _H), lambda i: (0, i, 0)),
            pl.BlockSpec((_BM, _H), lambda i: (i, 0)),
            pl.BlockSpec((_BM, _H), lambda i: (i, 0)),
            pl.BlockSpec((_BM, _NC), lambda i: (i, 0)),
            pl.BlockSpec((1, _D), lambda i: (0, 0)),
            pl.BlockSpec((_D, _D), lambda i: (0, 0)),
            pl.BlockSpec((1, _D), lambda i: (0, 0)),
        ],
        out_specs=[
            pl.BlockSpec((_BM, _D), lambda i: (i, 0)),
            pl.BlockSpec((1, _D), lambda i: (0, 0)),
        ],
        out_shape=[
            jax.ShapeDtypeStruct((_N, _D), jnp.float32),
            jax.ShapeDtypeStruct((1, _D), jnp.float32),
        ],
        scratch_shapes=[pltpu.VMEM((1, _D), jnp.float32)],
    )(seg, y0, y1, deg_parts, b1, wl, bl)


def kernel(graph_x, graph_edge, W1, b1, Wl, bl):
    src = graph_edge[0]
    dst = graph_edge[1]
    pad = _E_PAD - _E
    src_p = jnp.concatenate([src, jnp.zeros((pad,), jnp.int32)])
    dst_p = jnp.concatenate([dst, jnp.full((pad,), _N, jnp.int32)])
    dst2d = dst_p.reshape(_E_PAD // _CHUNK, _CHUNK)
    srcs = src_p.reshape(_E_PAD // _SCHUNK, _SCHUNK)
    dsts = dst_p.reshape(_E_PAD // _SCHUNK, _SCHUNK)

    zeros_n = jnp.zeros((_NPAD,), jnp.float32)
    ones_c = jnp.ones((_CHUNK,), jnp.float32)
    zrow = jnp.zeros((_ROWS_PER_TILE, _H), jnp.float32)

    xw = _mm_call(graph_x, W1)
    deg_parts = _deg_kernel(dst2d, zeros_n, ones_c).T
    y0, y1 = _y_call(xw, deg_parts)
    seg = _seg_kernel(y0, y1, srcs, dsts, zrow)
    h, out = _epi_call(seg, y0, y1, deg_parts,
                       b1.reshape(1, _D), Wl, bl.reshape(1, _D))
    return (h, out)


# speedup vs baseline: 10.6788x; 1.0002x over previous
import functools

import jax
import jax.numpy as jnp
from jax import lax
from jax.experimental import pallas as pl
from jax.experimental.pallas import tpu as pltpu
from jax.experimental.pallas import tpu_sc as plsc

_N = 10000
_E = 160000
_D = 256
_H = 128
_NPAD = 10240
_CHUNK = 128
_SCHUNK = 64

_NC = 2
_NS = 16
_NW = _NC * _NS

_E_PAD = 163840
_DEG_CH = _E_PAD // _NW // _CHUNK
_SEG_CH = _E_PAD // _NS // _SCHUNK
_SEG_PASSES = 4
_PASS_CH = _SEG_CH // _SEG_PASSES
_NBUF = 4
_ROWS_PER_TILE = _NPAD // _NS

_mesh = plsc.VectorSubcoreMesh(core_axis_name="c", subcore_axis_name="s")


@functools.partial(
    pl.kernel,
    mesh=_mesh,
    out_type=jax.ShapeDtypeStruct((_NC, _NPAD), jnp.float32),
    scratch_types=[
        pltpu.VMEM((_CHUNK,), jnp.float32),
        pltpu.VMEM((_DEG_CH, _CHUNK), jnp.int32),
        pltpu.VMEM_SHARED((_NPAD,), jnp.float32),
        pltpu.SemaphoreType.DMA,
    ],
)
def _deg_kernel(dst_hbm, zeros_hbm, ones_hbm, out_hbm, ones_v, idx_v, hist_sh, sem):
    c = lax.axis_index("c")
    s = lax.axis_index("s")
    w = s * _NC + c
    pltpu.sync_copy(ones_hbm, ones_v)
    pltpu.sync_copy(dst_hbm.at[pl.ds(w * _DEG_CH, _DEG_CH)], idx_v)

    @pl.when(s == 0)
    def _():
        pltpu.sync_copy(zeros_hbm, hist_sh)

    plsc.subcore_barrier()

    def fire(j, carry):
        pltpu.async_copy(ones_v, hist_sh.at[idx_v.at[j]], sem, add=True)
        return carry

    lax.fori_loop(0, _DEG_CH, fire, 0)

    def drain(j, carry):
        pltpu.make_async_copy(ones_v, hist_sh.at[idx_v.at[0]], sem).wait()
        return carry

    lax.fori_loop(0, _DEG_CH, drain, 0)
    plsc.subcore_barrier()

    @pl.when(s == 0)
    def _():
        pltpu.sync_copy(hist_sh, out_hbm.at[c])


@functools.partial(
    pl.kernel,
    mesh=_mesh,
    out_type=jax.ShapeDtypeStruct((_NC, _NPAD, _H), jnp.float32),
    scratch_types=[
        pltpu.VMEM((_PASS_CH, _SCHUNK), jnp.int32),
        pltpu.VMEM((_PASS_CH, _SCHUNK), jnp.int32),
        pltpu.VMEM((_SCHUNK, _H), jnp.float32),
        pltpu.VMEM((_SCHUNK, _H), jnp.float32),
        pltpu.VMEM((_SCHUNK, _H), jnp.float32),
        pltpu.VMEM((_SCHUNK, _H), jnp.float32),
        pltpu.VMEM_SHARED((_NPAD, _H), jnp.float32),
        pltpu.SemaphoreType.DMA,
        pltpu.SemaphoreType.DMA,
        pltpu.SemaphoreType.DMA,
        pltpu.SemaphoreType.DMA,
    ],
)
def _seg_kernel(y0, y1, src_hbm, dst_hbm, zrow_hbm, out_hbm,
                src_v, dst_v, b0, b1, b2, b3, seg_sh,
                g0, g1, g2, g3):
    c = lax.axis_index("c")
    s = lax.axis_index("s")
    bufs = (b0, b1, b2, b3)
    gsems = (g0, g1, g2, g3)
    pltpu.sync_copy(zrow_hbm, seg_sh.at[pl.ds(s * _ROWS_PER_TILE, _ROWS_PER_TILE)])
    plsc.subcore_barrier()

    def run(y_hbm):
        def g_fire(j, buf, sem):
            pltpu.async_copy(y_hbm.at[src_v.at[j]], buf, sem)

        def g_wait(buf, sem):
            pltpu.make_async_copy(y_hbm.at[src_v.at[0]], buf, sem).wait()

        for p in range(_SEG_PASSES):
            base = s * _SEG_CH + p * _PASS_CH
            pltpu.sync_copy(src_hbm.at[pl.ds(base, _PASS_CH)], src_v)
            pltpu.sync_copy(dst_hbm.at[pl.ds(base, _PASS_CH)], dst_v)
            g_fire(0, b0, g0)
            g_fire(1, b1, g1)
            g_fire(2, b2, g2)

            def body(i, carry):
                for k in range(_NBUF):
                    jj = _NBUF * i + k
                    q = (k + 3) % _NBUF
                    jn = jj + 3

                    @pl.when(jn < _PASS_CH)
                    def _(jn=jn, q=q):
                        g_fire(jn, bufs[q], gsems[q])
                    g_wait(bufs[k], gsems[k])
                    pltpu.sync_copy(bufs[k], seg_sh.at[dst_v.at[jj]], add=True)
                return carry

            lax.fori_loop(0, _PASS_CH // _NBUF, body, 0)

    @pl.when(c == 0)
    def _():
        run(y0)

    @pl.when(c == 1)
    def _():
        run(y1)

    plsc.subcore_barrier()
    pltpu.sync_copy(
        seg_sh.at[pl.ds(s * _ROWS_PER_TILE, _ROWS_PER_TILE)],
        out_hbm.at[c, pl.ds(s * _ROWS_PER_TILE, _ROWS_PER_TILE)],
    )


_BM = 1000


def _mm_body(x_ref, w_ref, xw_ref):
    xw_ref[...] = jnp.dot(x_ref[...], w_ref[...],
                          preferred_element_type=jnp.float32)


def _mm_call(x, w1):
    return pl.pallas_call(
        _mm_body,
        grid=(_N // _BM,),
        in_specs=[
            pl.BlockSpec((_BM, _D), lambda i: (i, 0)),
            pl.BlockSpec((_D, _D), lambda i: (0, 0)),
        ],
        out_specs=pl.BlockSpec((_BM, _D), lambda i: (i, 0)),
        out_shape=jax.ShapeDtypeStruct((_N, _D), jnp.float32),
    )(x, w1)


def _y_body(xw_ref, dp_ref, y0_ref, y1_ref):
    deg = dp_ref[:, 0] + dp_ref[:, 1] + 1.0
    dinv = lax.rsqrt(deg)[:, None]
    y = xw_ref[...] * dinv
    y0_ref[...] = y[:, :_H]
    y1_ref[...] = y[:, _H:]


def _y_call(xw, deg_parts):
    return pl.pallas_call(
        _y_body,
        grid=(_N // _BM,),
        in_specs=[
            pl.BlockSpec((_BM, _D), lambda i: (i, 0)),
            pl.BlockSpec((_BM, _NC), lambda i: (i, 0)),
        ],
        out_specs=[
            pl.BlockSpec((_BM, _H), lambda i: (i, 0)),
            pl.BlockSpec((_BM, _H), lambda i: (i, 0)),
        ],
        out_shape=[
            jax.ShapeDtypeStruct((_N, _H), jnp.float32),
            jax.ShapeDtypeStruct((_N, _H), jnp.float32),
        ],
    )(xw, deg_parts)


def _epi_body(seg_ref, y0_ref, y1_ref, dp_ref, b1_ref, wl_ref, bl_ref,
              h_ref, out_ref, acc_ref):
    i = pl.program_id(0)
    deg = dp_ref[:, 0] + dp_ref[:, 1] + 1.0
    dinv = lax.rsqrt(deg)[:, None]
    h0 = jnp.maximum(dinv * (seg_ref[0] + y0_ref[...]) + b1_ref[0, :_H], 0.0)
    h1 = jnp.maximum(dinv * (seg_ref[1] + y1_ref[...]) + b1_ref[0, _H:], 0.0)
    h_ref[:, :_H] = h0
    h_ref[:, _H:] = h1

    @pl.when(i == 0)
    def _():
        acc_ref[...] = jnp.zeros_like(acc_ref)

    acc_ref[0, :_H] += jnp.sum(h0, axis=0)
    acc_ref[0, _H:] += jnp.sum(h1, axis=0)

    @pl.when(i == pl.num_programs(0) - 1)
    def _():
        mean = acc_ref[...] * (1.0 / _N)
        out_ref[...] = lax.dot_general(
            mean, wl_ref[...], (((1,), (1,)), ((), ()))) + bl_ref[...]


def _epi_call(seg, y0, y1, deg_parts, b1, wl, bl):
    return pl.pallas_call(
        _epi_body,
        grid=(_N // _BM,),
        in_specs=[
            pl.BlockSpec((_NC, _BM, _H), lambda i: (0, i, 0)),
            pl.BlockSpec((_BM, _H), lambda i: (i, 0)),
            pl.BlockSpec((_BM, _H), lambda i: (i, 0)),
            pl.BlockSpec((_BM, _NC), lambda i: (i, 0)),
            pl.BlockSpec((1, _D), lambda i: (0, 0)),
            pl.BlockSpec((_D, _D), lambda i: (0, 0)),
            pl.BlockSpec((1, _D), lambda i: (0, 0)),
        ],
        out_specs=[
            pl.BlockSpec((_BM, _D), lambda i: (i, 0)),
            pl.BlockSpec((1, _D), lambda i: (0, 0)),
        ],
        out_shape=[
            jax.ShapeDtypeStruct((_N, _D), jnp.float32),
            jax.ShapeDtypeStruct((1, _D), jnp.float32),
        ],
        scratch_shapes=[pltpu.VMEM((1, _D), jnp.float32)],
    )(seg, y0, y1, deg_parts, b1, wl, bl)


def kernel(graph_x, graph_edge, W1, b1, Wl, bl):
    src = graph_edge[0]
    dst = graph_edge[1]
    pad = _E_PAD - _E
    src_p = jnp.concatenate([src, jnp.zeros((pad,), jnp.int32)])
    dst_p = jnp.concatenate([dst, jnp.full((pad,), _N, jnp.int32)])
    dst2d = dst_p.reshape(_E_PAD // _CHUNK, _CHUNK)
    srcs = src_p.reshape(_E_PAD // _SCHUNK, _SCHUNK)
    dsts = dst_p.reshape(_E_PAD // _SCHUNK, _SCHUNK)

    zeros_n = jnp.zeros((_NPAD,), jnp.float32)
    ones_c = jnp.ones((_CHUNK,), jnp.float32)
    zrow = jnp.zeros((_ROWS_PER_TILE, _H), jnp.float32)

    xw = _mm_call(graph_x, W1)
    deg_parts = _deg_kernel(dst2d, zeros_n, ones_c).T
    y0, y1 = _y_call(xw, deg_parts)
    seg = _seg_kernel(y0, y1, srcs, dsts, zrow)
    h, out = _epi_call(seg, y0, y1, deg_parts,
                       b1.reshape(1, _D), Wl, bl.reshape(1, _D))
    return (h, out)
